# baseline (speedup 1.0000x reference)
import functools

import jax
import jax.numpy as jnp
from jax import lax
from jax.experimental import pallas as pl
from jax.experimental.pallas import tpu as pltpu
from jax.experimental.pallas import tpu_sc as plsc

BOHR_AU = 1.8897261258369282
N_PAIRS = 6400000
SPECIES = 90
N_POLY = 12

NUM_CORES = 2
NUM_SUBCORES = 16
LANES = 16

ROWS = 50000
COLS = 128
CH_R = 500
NCH = ROWS // CH_R
NBUF = 10


def _sc_gather_row(spec16, param_pad):
    mesh = plsc.VectorSubcoreMesh(core_axis_name="c", subcore_axis_name="s")

    @functools.partial(
        pl.kernel,
        out_type=jax.ShapeDtypeStruct((LANES,), jnp.float32),
        mesh=mesh,
        compiler_params=pltpu.CompilerParams(needs_layout_passes=False),
        scratch_types=[
            pltpu.VMEM((LANES,), jnp.int32),
            pltpu.VMEM((LANES,), jnp.float32),
            pltpu.SemaphoreType.DMA,
        ],
    )
    def gather_kernel(spec_hbm, param_hbm, out_hbm, spec_v, row_v, sem):
        wid = lax.axis_index("c") * NUM_SUBCORES + lax.axis_index("s")

        @pl.when(wid == 0)
        def _():
            pltpu.sync_copy(spec_hbm, spec_v)
            sv = spec_v[...]
            flat = sv[0] * SPECIES + sv[1]
            pltpu.async_copy(
                param_hbm.at[pl.ds(flat * LANES, LANES)], row_v, sem).wait()
            pltpu.sync_copy(row_v, out_hbm)

    return gather_kernel(spec16, param_pad)


def _tc_horner(x2d, row):
    def body(row_ref, x_hbm, o_hbm, xb, yb, sem_in, sem_out):
        cs = [row_ref[k] * jnp.float32(BOHR_AU ** k) for k in range(N_POLY)]

        H = CH_R // 2

        class _Pair:
            def __init__(self, d0, d1):
                self._d = (d0, d1)

            def start(self):
                self._d[0].start()
                self._d[1].start()

            def wait(self):
                self._d[0].wait()
                self._d[1].wait()

        def in_dma(b, c):
            r = c * CH_R
            return _Pair(
                pltpu.make_async_copy(
                    x_hbm.at[pl.ds(r, H)], xb.at[b, pl.ds(0, H)],
                    sem_in.at[b]),
                pltpu.make_async_copy(
                    x_hbm.at[pl.ds(r + H, H)], xb.at[b, pl.ds(H, H)],
                    sem_in.at[b]))

        def out_dma(b, c):
            r = c * CH_R
            return _Pair(
                pltpu.make_async_copy(
                    yb.at[b, pl.ds(0, H)], o_hbm.at[pl.ds(r, H)],
                    sem_out.at[b]),
                pltpu.make_async_copy(
                    yb.at[b, pl.ds(H, H)], o_hbm.at[pl.ds(r + H, H)],
                    sem_out.at[b]))

        for b in range(NBUF):
            in_dma(b, b).start()
        for c in range(NCH):
            b = c % NBUF
            in_dma(b, c).wait()
            if c >= NBUF:
                out_dma(b, c - NBUF).wait()
            x = xb[b]
            y = jnp.full(x.shape, cs[N_POLY - 1])
            for k in range(N_POLY - 2, -1, -1):
                y = y * x + cs[k]
            yb[b] = y
            out_dma(b, c).start()
            if c + NBUF < NCH:
                in_dma(b, c + NBUF).start()
        for c in range(NCH - NBUF, NCH):
            out_dma(c % NBUF, c).wait()

    return pl.pallas_call(
        body,
        in_specs=[
            pl.BlockSpec(memory_space=pltpu.SMEM),
            pl.BlockSpec(memory_space=pltpu.HBM),
        ],
        out_specs=pl.BlockSpec(memory_space=pltpu.HBM),
        out_shape=jax.ShapeDtypeStruct((ROWS, COLS), jnp.float32),
        scratch_shapes=[
            pltpu.VMEM((NBUF, CH_R, COLS), jnp.float32),
            pltpu.VMEM((NBUF, CH_R, COLS), jnp.float32),
            pltpu.SemaphoreType.DMA((NBUF,)),
            pltpu.SemaphoreType.DMA((NBUF,)),
        ],
    )(row, x2d)


def kernel(dr, species_tuple, param_matrix):
    spec16 = jnp.zeros((LANES,), jnp.int32).at[:2].set(
        species_tuple.astype(jnp.int32))
    param_pad = jnp.pad(
        param_matrix.reshape(SPECIES * SPECIES, N_POLY),
        ((0, 0), (0, LANES - N_POLY))).reshape(-1)
    row = _sc_gather_row(spec16, param_pad)
    y2d = _tc_horner(dr.reshape(ROWS, COLS), row)
    return y2d.reshape(-1)


# speedup vs baseline: 1.0013x; 1.0013x over previous
import functools

import jax
import jax.numpy as jnp
from jax import lax
from jax.experimental import pallas as pl
from jax.experimental.pallas import tpu as pltpu
from jax.experimental.pallas import tpu_sc as plsc

BOHR_AU = 1.8897261258369282
N_PAIRS = 6400000
SPECIES = 90
N_POLY = 12

NUM_CORES = 2
NUM_SUBCORES = 16
LANES = 16

ROWS = 50000
COLS = 128
CH_R = 500
NCH = ROWS // CH_R
NBUF = 10


def _sc_gather_row(spec16, param_pad):
    mesh = plsc.VectorSubcoreMesh(core_axis_name="c", subcore_axis_name="s")

    @functools.partial(
        pl.kernel,
        out_type=jax.ShapeDtypeStruct((LANES,), jnp.float32),
        mesh=mesh,
        compiler_params=pltpu.CompilerParams(needs_layout_passes=False),
        scratch_types=[
            pltpu.VMEM((LANES,), jnp.int32),
            pltpu.VMEM((LANES,), jnp.float32),
            pltpu.SemaphoreType.DMA,
        ],
    )
    def gather_kernel(spec_hbm, param_hbm, out_hbm, spec_v, row_v, sem):
        wid = lax.axis_index("c") * NUM_SUBCORES + lax.axis_index("s")

        @pl.when(wid == 0)
        def _():
            pltpu.sync_copy(spec_hbm, spec_v)
            sv = spec_v[...]
            flat = sv[0] * SPECIES + sv[1]
            pltpu.async_copy(
                param_hbm.at[pl.ds(flat * LANES, LANES)], row_v, sem).wait()
            pltpu.sync_copy(row_v, out_hbm)

    return gather_kernel(spec16, param_pad)


def _tc_horner(x2d, row):
    def body(row_ref, x_hbm, o_hbm, xb, yb, sem_in, sem_out):
        cs = [row_ref[k] * jnp.float32(BOHR_AU ** k) for k in range(N_POLY)]

        def in_dma(b, c):
            return pltpu.make_async_copy(
                x_hbm.at[pl.ds(c * CH_R, CH_R)], xb.at[b], sem_in.at[b])

        def out_dma(b, c):
            return pltpu.make_async_copy(
                yb.at[b], o_hbm.at[pl.ds(c * CH_R, CH_R)], sem_out.at[b])

        for b in range(NBUF):
            in_dma(b, b).start()
        for c in range(NCH):
            b = c % NBUF
            in_dma(b, c).wait()
            if c >= NBUF:
                out_dma(b, c - NBUF).wait()
            x = xb[b]
            y = jnp.full(x.shape, cs[N_POLY - 1])
            for k in range(N_POLY - 2, -1, -1):
                y = y * x + cs[k]
            yb[b] = y
            out_dma(b, c).start()
            if c + NBUF < NCH:
                in_dma(b, c + NBUF).start()
        for c in range(NCH - NBUF, NCH):
            out_dma(c % NBUF, c).wait()

    return pl.pallas_call(
        body,
        in_specs=[
            pl.BlockSpec(memory_space=pltpu.SMEM),
            pl.BlockSpec(memory_space=pltpu.HBM),
        ],
        out_specs=pl.BlockSpec(memory_space=pltpu.HBM),
        out_shape=jax.ShapeDtypeStruct((ROWS, COLS), jnp.float32),
        scratch_shapes=[
            pltpu.VMEM((NBUF, CH_R, COLS), jnp.float32),
            pltpu.VMEM((NBUF, CH_R, COLS), jnp.float32),
            pltpu.SemaphoreType.DMA((NBUF,)),
            pltpu.SemaphoreType.DMA((NBUF,)),
        ],
    )(row, x2d)


def kernel(dr, species_tuple, param_matrix):
    spec16 = jnp.zeros((LANES,), jnp.int32).at[:2].set(
        species_tuple.astype(jnp.int32))
    param_pad = jnp.pad(
        param_matrix.reshape(SPECIES * SPECIES, N_POLY),
        ((0, 0), (0, LANES - N_POLY))).reshape(-1)
    row = _sc_gather_row(spec16, param_pad)
    y2d = _tc_horner(dr.reshape(ROWS, COLS), row)
    return y2d.reshape(-1)
